# Initial kernel scaffold; baseline (speedup 1.0000x reference)
import functools

import jax
import jax.numpy as jnp
from jax import lax
from jax.experimental import pallas as pl
from jax.experimental.pallas import tpu as pltpu
from jax.experimental.pallas import tpu_sc as plsc

N = 10000
E = 320000
D = 128
NHID = 8
NHEAD = 8
NCLASS = 16
ALPHA = 0.2

NCORE = 2
NSUB = 16
NW = NCORE * NSUB
EPW = E // NW
CHUNK = 80
NCHUNK = EPW // CHUNK
ROWS_PT = N // NSUB

_i32 = jnp.int32
_f32 = jnp.float32


def _iota16():
    return lax.iota(_i32, 16)


def _splat(v):
    return jnp.zeros((16,), _i32) + v


def _dense1_body(x_ref, w_ref, a_ref, h_ref, asat_ref, mx_ref):
    i = pl.program_id(0)
    hb = jnp.dot(x_ref[...], w_ref[...], preferred_element_type=_f32)
    asb = jnp.dot(hb, a_ref[...], preferred_element_type=_f32)
    h_ref[...] = hb
    asat_ref[...] = asb
    bm = jnp.broadcast_to(jnp.max(asb, axis=0, keepdims=True), (8, 16))

    @pl.when(i == 0)
    def _():
        mx_ref[...] = bm

    @pl.when(i > 0)
    def _():
        mx_ref[...] = jnp.maximum(mx_ref[...], bm)


def _dense1(x, w1cat, a1cat, blk=2000):
    return pl.pallas_call(
        _dense1_body,
        grid=(N // blk,),
        in_specs=[
            pl.BlockSpec((blk, D), lambda i: (i, 0)),
            pl.BlockSpec((D, NHEAD * NHID), lambda i: (0, 0)),
            pl.BlockSpec((NHEAD * NHID, 16), lambda i: (0, 0)),
        ],
        out_specs=[
            pl.BlockSpec((blk, NHEAD * NHID), lambda i: (i, 0)),
            pl.BlockSpec((blk, 16), lambda i: (i, 0)),
            pl.BlockSpec((8, 16), lambda i: (0, 0)),
        ],
        out_shape=[
            jax.ShapeDtypeStruct((N, NHEAD * NHID), _f32),
            jax.ShapeDtypeStruct((N, 16), _f32),
            jax.ShapeDtypeStruct((8, 16), _f32),
        ],
        name="dense1",
    )(x, w1cat, a1cat)


def _rsum_body(s_ref, out_ref):
    r = 1.0 / (s_ref[0] + s_ref[1] + 1e-10)
    out_ref[...] = jnp.concatenate([r, r], axis=1)


def _rsum(sumpart, blk=2000):
    return pl.pallas_call(
        _rsum_body,
        grid=(N // blk,),
        in_specs=[pl.BlockSpec((2, blk, 8), lambda i: (0, i, 0))],
        out_specs=pl.BlockSpec((blk, 16), lambda i: (i, 0)),
        out_shape=jax.ShapeDtypeStruct((N, 16), _f32),
        name="rsum",
    )(sumpart)


def _dense2_body(op_ref, b_ref, w_ref, a_ref, h2_ref, asat_ref, mx_ref):
    i = pl.program_id(0)
    hp = op_ref[0] + op_ref[1] + b_ref[...]
    h = jnp.where(hp > 0, hp, jnp.expm1(hp))
    h2 = jnp.dot(h, w_ref[...], preferred_element_type=_f32)
    asb = jnp.dot(h2, a_ref[...], preferred_element_type=_f32)
    h2_ref[...] = h2
    asat_ref[...] = asb
    bm = jnp.broadcast_to(jnp.max(asb, axis=0, keepdims=True), (8, 16))

    @pl.when(i == 0)
    def _():
        mx_ref[...] = bm

    @pl.when(i > 0)
    def _():
        mx_ref[...] = jnp.maximum(mx_ref[...], bm)


def _dense2(outpart, b1mat, w2, a2pad, blk=2000):
    return pl.pallas_call(
        _dense2_body,
        grid=(N // blk,),
        in_specs=[
            pl.BlockSpec((2, blk, 64), lambda i: (0, i, 0)),
            pl.BlockSpec((1, 64), lambda i: (0, 0)),
            pl.BlockSpec((64, NCLASS), lambda i: (0, 0)),
            pl.BlockSpec((NCLASS, 16), lambda i: (0, 0)),
        ],
        out_specs=[
            pl.BlockSpec((blk, NCLASS), lambda i: (i, 0)),
            pl.BlockSpec((blk, 16), lambda i: (i, 0)),
            pl.BlockSpec((8, 16), lambda i: (0, 0)),
        ],
        out_shape=[
            jax.ShapeDtypeStruct((N, NCLASS), _f32),
            jax.ShapeDtypeStruct((N, 16), _f32),
            jax.ShapeDtypeStruct((8, 16), _f32),
        ],
        name="dense2",
    )(outpart, b1mat, w2, a2pad)


def _final_body(op_ref, b_ref, out_ref):
    v = op_ref[0] + op_ref[1] + b_ref[...]
    m = jnp.max(v, axis=1, keepdims=True)
    ex = jnp.exp(v - m)
    out_ref[...] = (v - m) - jnp.log(jnp.sum(ex, axis=1, keepdims=True))


def _final(outpart2, b2mat, blk=2000):
    return pl.pallas_call(
        _final_body,
        grid=(N // blk,),
        in_specs=[
            pl.BlockSpec((2, blk, NCLASS), lambda i: (0, i, 0)),
            pl.BlockSpec((1, NCLASS), lambda i: (0, 0)),
        ],
        out_specs=pl.BlockSpec((blk, NCLASS), lambda i: (i, 0)),
        out_shape=jax.ShapeDtypeStruct((N, NCLASS), _f32),
        name="final_logsoftmax",
    )(outpart2, b2mat)


_MESH = plsc.VectorSubcoreMesh(core_axis_name="c", subcore_axis_name="s")


def _wid_base():
    c = lax.axis_index("c")
    s = lax.axis_index("s")
    return c, s, (s * NCORE + c) * EPW


def _edge_softmax_body(nheads, src_hbm, tgt_hbm, asat_hbm, ub_hbm, z8_hbm,
                       exp_hbm, sum_hbm,
                       src_v, tgt_v, srows, trows, expb, ubv, shared_sum):
    c, s, ebase = _wid_base()
    n0 = s * ROWS_PT
    pltpu.sync_copy(ub_hbm, ubv)
    if nheads < 8:
        pltpu.sync_copy(z8_hbm.at[pl.ds(0, CHUNK)], expb)
    pltpu.sync_copy(z8_hbm.at[pl.ds(n0, ROWS_PT)],
                    shared_sum.at[pl.ds(n0, ROWS_PT)])
    plsc.subcore_barrier()

    def chunk(t, _):
        e0 = ebase + t * CHUNK
        pltpu.sync_copy(src_hbm.at[pl.ds(e0, CHUNK)], src_v)
        pltpu.sync_copy(tgt_hbm.at[pl.ds(e0, CHUNK)], tgt_v)
        pltpu.sync_copy(asat_hbm.at[src_v], srows)
        pltpu.sync_copy(asat_hbm.at[tgt_v], trows)

        def kb(k, _):
            rows = _iota16() + k * 16
            for h in range(nheads):
                s16 = plsc.load_gather(srows, [rows, _splat(h)])
                t16 = plsc.load_gather(trows, [rows, _splat(h + NHEAD)])
                e16 = s16 + t16
                e16 = jnp.maximum(e16, ALPHA * e16)
                ubh = plsc.load_gather(ubv, [_splat(h)])
                e16 = jnp.exp(e16 - ubh)
                plsc.store_scatter(expb, [rows, _splat(h)], e16)
            return 0

        lax.fori_loop(0, CHUNK // 16, kb, 0)
        pltpu.sync_copy(expb, exp_hbm.at[pl.ds(e0, CHUNK)])
        pltpu.sync_copy(expb, shared_sum.at[src_v], add=True)
        return 0

    lax.fori_loop(0, NCHUNK, chunk, 0)
    plsc.subcore_barrier()
    pltpu.sync_copy(shared_sum.at[pl.ds(n0, ROWS_PT)],
                    sum_hbm.at[c, pl.ds(n0, ROWS_PT)])


def _edge_softmax(nheads, src, tgt, asat, ubpad, z8):
    body = functools.partial(_edge_softmax_body, nheads)
    k = pl.kernel(
        body,
        out_type=[
            jax.ShapeDtypeStruct((E, 8), _f32),
            jax.ShapeDtypeStruct((2, N, 8), _f32),
        ],
        mesh=_MESH,
        scratch_types=[
            pltpu.VMEM((CHUNK,), _i32),
            pltpu.VMEM((CHUNK,), _i32),
            pltpu.VMEM((CHUNK, 16), _f32),
            pltpu.VMEM((CHUNK, 16), _f32),
            pltpu.VMEM((CHUNK, 8), _f32),
            pltpu.VMEM((16,), _f32),
            pltpu.VMEM_SHARED((N, 8), _f32),
        ],
        name=f"sc_edge_softmax_h{nheads}",
    )
    return k(src, tgt, asat, ubpad, z8)


def _edge_aggregate_body(nf, src_hbm, tgt_hbm, exp_hbm, rsum_hbm, h_hbm,
                         z_hbm, out_hbm,
                         src_v, tgt_v, ht, rs, expb, attb, shared_out):
    nheads = nf // NHID if nf == 64 else 1
    lg = {8: 3, 16: 4}[nf // nheads]
    c, s, ebase = _wid_base()
    n0 = s * ROWS_PT
    pltpu.sync_copy(z_hbm.at[pl.ds(n0, ROWS_PT)],
                    shared_out.at[pl.ds(n0, ROWS_PT)])
    plsc.subcore_barrier()

    qcols = [lax.shift_right_logical(_iota16() + 16 * q, lg)
             for q in range(nf // 16)]

    def chunk(t, _):
        e0 = ebase + t * CHUNK
        pltpu.sync_copy(src_hbm.at[pl.ds(e0, CHUNK)], src_v)
        pltpu.sync_copy(tgt_hbm.at[pl.ds(e0, CHUNK)], tgt_v)
        pltpu.sync_copy(h_hbm.at[tgt_v], ht)
        pltpu.sync_copy(rsum_hbm.at[src_v], rs)
        pltpu.sync_copy(exp_hbm.at[pl.ds(e0, CHUNK)], expb)

        def kb(k, _):
            rows = _iota16() + k * 16
            for h in range(nheads):
                a16 = (plsc.load_gather(expb, [rows, _splat(h)])
                       * plsc.load_gather(rs, [rows, _splat(h)]))
                plsc.store_scatter(attb, [rows, _splat(h)], a16)
            return 0

        lax.fori_loop(0, CHUNK // 16, kb, 0)

        def eb(i, _):
            ri = _splat(i)
            for q in range(nf // 16):
                a16 = plsc.load_gather(attb, [ri, qcols[q]])
                ht[i, pl.ds(16 * q, 16)] = a16 * ht[i, pl.ds(16 * q, 16)]
            return 0

        lax.fori_loop(0, CHUNK, eb, 0)
        pltpu.sync_copy(ht, shared_out.at[src_v], add=True)
        return 0

    lax.fori_loop(0, NCHUNK, chunk, 0)
    plsc.subcore_barrier()
    pltpu.sync_copy(shared_out.at[pl.ds(n0, ROWS_PT)],
                    out_hbm.at[c, pl.ds(n0, ROWS_PT)])


def _edge_aggregate(nf, src, tgt, expv, rsum, h, z):
    body = functools.partial(_edge_aggregate_body, nf)
    k = pl.kernel(
        body,
        out_type=jax.ShapeDtypeStruct((2, N, nf), _f32),
        mesh=_MESH,
        scratch_types=[
            pltpu.VMEM((CHUNK,), _i32),
            pltpu.VMEM((CHUNK,), _i32),
            pltpu.VMEM((CHUNK, nf), _f32),
            pltpu.VMEM((CHUNK, 16), _f32),
            pltpu.VMEM((CHUNK, 8), _f32),
            pltpu.VMEM((CHUNK, 8), _f32),
            pltpu.VMEM_SHARED((N, nf), _f32),
        ],
        name=f"sc_edge_aggregate_f{nf}",
    )
    return k(src, tgt, expv, rsum, h, z)


def _ub_from_mx(mx):
    m = jnp.max(mx, axis=0)
    ssum = m[:8] + m[8:]
    ub = jnp.maximum(ssum, ALPHA * ssum)
    return jnp.concatenate([ub, jnp.zeros((8,), _f32)])


def kernel(x, edge_index, W1, a1, b1, W2, a2, b2):
    src = edge_index[0]
    tgt = edge_index[1]

    w1cat = jnp.transpose(W1, (1, 0, 2)).reshape(D, NHEAD * NHID)
    eye8 = jnp.eye(NHEAD, dtype=_f32)
    a_src = (a1[:, :NHID, 0][:, :, None] * eye8[:, None, :]).reshape(
        NHEAD * NHID, NHEAD)
    a_tgt = (a1[:, NHID:, 0][:, :, None] * eye8[:, None, :]).reshape(
        NHEAD * NHID, NHEAD)
    a1cat = jnp.concatenate([a_src, a_tgt], axis=1)
    b1mat = b1[:, 0, :].reshape(1, NHEAD * NHID)
    w2m = W2[0]
    a2pad = jnp.concatenate(
        [a2[0, :NCLASS], a2[0, NCLASS:],
         jnp.zeros((NCLASS, 14), _f32)], axis=1)
    b2mat = b2[0]

    z8 = jnp.zeros((N, 8), _f32)
    z16 = jnp.zeros((N, 16), _f32)
    z64 = jnp.zeros((N, 64), _f32)

    h1, asat1, mx1 = _dense1(x, w1cat, a1cat)
    ub1 = _ub_from_mx(mx1)
    exp1, sum1p = _edge_softmax(NHEAD, src, tgt, asat1, ub1, z8)
    rsum1 = _rsum(sum1p)
    out1p = _edge_aggregate(64, src, tgt, exp1, rsum1, h1, z64)

    h2, asat2, mx2 = _dense2(out1p, b1mat, w2m, a2pad)
    ub2 = _ub_from_mx(mx2)
    exp2, sum2p = _edge_softmax(1, src, tgt, asat2, ub2, z8)
    rsum2 = _rsum(sum2p)
    out2p = _edge_aggregate(16, src, tgt, exp2, rsum2, h2, z16)

    return _final(out2p, b2mat)


# speedup vs baseline: 19.9361x; 19.9361x over previous
import functools

import jax
import jax.numpy as jnp
from jax import lax
from jax.experimental import pallas as pl
from jax.experimental.pallas import tpu as pltpu
from jax.experimental.pallas import tpu_sc as plsc

N = 10000
E = 320000
D = 128
NHID = 8
NHEAD = 8
NCLASS = 16
ALPHA = 0.2

NCORE = 2
NSUB = 16
NW = NCORE * NSUB
EPW = E // NW
CHUNK = 80
NCHUNK = EPW // CHUNK
NPAD = 10240
ROWS_PT = NPAD // NSUB

_i32 = jnp.int32
_f32 = jnp.float32


def _iota16():
    return lax.iota(_i32, 16)


def _splat(v):
    return jnp.zeros((16,), _i32) + v


def _dense1_body(x_ref, w_ref, a_ref, h_ref, asat_ref, mx_ref):
    i = pl.program_id(0)
    hb = jnp.dot(x_ref[...], w_ref[...], preferred_element_type=_f32)
    asb = jnp.dot(hb, a_ref[...], preferred_element_type=_f32)
    h_ref[...] = hb
    asat_ref[...] = asb
    bm = jnp.broadcast_to(jnp.max(asb, axis=0, keepdims=True), (8, 16))

    @pl.when(i == 0)
    def _():
        mx_ref[...] = bm

    @pl.when(i > 0)
    def _():
        mx_ref[...] = jnp.maximum(mx_ref[...], bm)


def _dense1(x, w1cat, a1cat, blk=2000):
    return pl.pallas_call(
        _dense1_body,
        grid=(N // blk,),
        in_specs=[
            pl.BlockSpec((blk, D), lambda i: (i, 0)),
            pl.BlockSpec((D, NHEAD * NHID), lambda i: (0, 0)),
            pl.BlockSpec((NHEAD * NHID, 16), lambda i: (0, 0)),
        ],
        out_specs=[
            pl.BlockSpec((blk, NHEAD * NHID), lambda i: (i, 0)),
            pl.BlockSpec((blk, 16), lambda i: (i, 0)),
            pl.BlockSpec((8, 16), lambda i: (0, 0)),
        ],
        out_shape=[
            jax.ShapeDtypeStruct((N, NHEAD * NHID), _f32),
            jax.ShapeDtypeStruct((N, 16), _f32),
            jax.ShapeDtypeStruct((8, 16), _f32),
        ],
        name="dense1",
    )(x, w1cat, a1cat)


def _rsum_body(s_ref, out_ref):
    r = 1.0 / (s_ref[0] + s_ref[1] + 1e-10)
    out_ref[...] = jnp.concatenate([r, r], axis=1)


def _rsum(sumpart, blk=2048):
    return pl.pallas_call(
        _rsum_body,
        grid=(NPAD // blk,),
        in_specs=[pl.BlockSpec((2, blk, 8), lambda i: (0, i, 0))],
        out_specs=pl.BlockSpec((blk, 16), lambda i: (i, 0)),
        out_shape=jax.ShapeDtypeStruct((NPAD, 16), _f32),
        name="rsum",
    )(sumpart)


def _dense2_body(op_ref, b_ref, w_ref, a_ref, h2_ref, asat_ref, mx_ref):
    i = pl.program_id(0)
    hp = op_ref[0] + op_ref[1] + b_ref[...]
    h = jnp.where(hp > 0, hp, jnp.exp(hp) - 1.0)
    h2 = jnp.dot(h, w_ref[...], preferred_element_type=_f32)
    asb = jnp.dot(h2, a_ref[...], preferred_element_type=_f32)
    h2_ref[...] = h2
    asat_ref[...] = asb
    bm = jnp.broadcast_to(jnp.max(asb, axis=0, keepdims=True), (8, 16))

    @pl.when(i == 0)
    def _():
        mx_ref[...] = bm

    @pl.when(i > 0)
    def _():
        mx_ref[...] = jnp.maximum(mx_ref[...], bm)


def _dense2(outpart, b1mat, w2, a2pad, blk=1024):
    return pl.pallas_call(
        _dense2_body,
        grid=(NPAD // blk,),
        in_specs=[
            pl.BlockSpec((2, blk, 64), lambda i: (0, i, 0)),
            pl.BlockSpec((1, 64), lambda i: (0, 0)),
            pl.BlockSpec((64, NCLASS), lambda i: (0, 0)),
            pl.BlockSpec((NCLASS, 16), lambda i: (0, 0)),
        ],
        out_specs=[
            pl.BlockSpec((blk, NCLASS), lambda i: (i, 0)),
            pl.BlockSpec((blk, 16), lambda i: (i, 0)),
            pl.BlockSpec((8, 16), lambda i: (0, 0)),
        ],
        out_shape=[
            jax.ShapeDtypeStruct((NPAD, NCLASS), _f32),
            jax.ShapeDtypeStruct((NPAD, 16), _f32),
            jax.ShapeDtypeStruct((8, 16), _f32),
        ],
        name="dense2",
    )(outpart, b1mat, w2, a2pad)


def _final_body(op_ref, b_ref, out_ref):
    v = op_ref[0] + op_ref[1] + b_ref[...]
    m = jnp.max(v, axis=1, keepdims=True)
    ex = jnp.exp(v - m)
    out_ref[...] = (v - m) - jnp.log(jnp.sum(ex, axis=1, keepdims=True))


def _final(outpart2, b2mat, blk=1024):
    return pl.pallas_call(
        _final_body,
        grid=(NPAD // blk,),
        in_specs=[
            pl.BlockSpec((2, blk, NCLASS), lambda i: (0, i, 0)),
            pl.BlockSpec((1, NCLASS), lambda i: (0, 0)),
        ],
        out_specs=pl.BlockSpec((blk, NCLASS), lambda i: (i, 0)),
        out_shape=jax.ShapeDtypeStruct((NPAD, NCLASS), _f32),
        name="final_logsoftmax",
    )(outpart2, b2mat)


@functools.cache
def _mesh():
    return plsc.VectorSubcoreMesh(core_axis_name="c", subcore_axis_name="s",
                                  num_cores=NCORE, num_subcores=NSUB)


def _wid_base():
    c = lax.axis_index("c")
    s = lax.axis_index("s")
    return c, s, (s * NCORE + c) * EPW


def _edge_softmax_body(nheads, src_hbm, tgt_hbm, asat_hbm, ub_hbm, z8_hbm,
                       exp_hbm, sum_hbm,
                       src_v, tgt_v, srows, trows, expb, ubv, shared_sum):
    c, s, ebase = _wid_base()
    n0 = s * ROWS_PT
    pltpu.sync_copy(ub_hbm, ubv)
    if nheads < 8:
        pltpu.sync_copy(z8_hbm.at[pl.ds(0, CHUNK)], expb)
    pltpu.sync_copy(z8_hbm.at[pl.ds(n0, ROWS_PT)],
                    shared_sum.at[pl.ds(n0, ROWS_PT)])
    plsc.subcore_barrier()

    def chunk(t, _):
        e0 = ebase + t * CHUNK
        pltpu.sync_copy(src_hbm.at[pl.ds(e0, CHUNK)], src_v)
        pltpu.sync_copy(tgt_hbm.at[pl.ds(e0, CHUNK)], tgt_v)
        pltpu.sync_copy(asat_hbm.at[src_v], srows)
        pltpu.sync_copy(asat_hbm.at[tgt_v], trows)

        def kb(k, _):
            rows = _iota16() + k * 16
            for h in range(nheads):
                s16 = plsc.load_gather(srows, [rows, _splat(h)])
                t16 = plsc.load_gather(trows, [rows, _splat(h + NHEAD)])
                e16 = s16 + t16
                e16 = jnp.maximum(e16, ALPHA * e16)
                ubh = plsc.load_gather(ubv, [_splat(h)])
                e16 = jnp.exp(e16 - ubh)
                plsc.store_scatter(expb, [rows, _splat(h)], e16)
            return 0

        lax.fori_loop(0, CHUNK // 16, kb, 0)
        pltpu.sync_copy(expb, exp_hbm.at[pl.ds(e0, CHUNK)])
        pltpu.sync_copy(expb, shared_sum.at[src_v], add=True)
        return 0

    lax.fori_loop(0, NCHUNK, chunk, 0)
    plsc.subcore_barrier()
    pltpu.sync_copy(shared_sum.at[pl.ds(n0, ROWS_PT)],
                    sum_hbm.at[c, pl.ds(n0, ROWS_PT)])


def _edge_softmax(nheads, src, tgt, asat, ubpad, z8):
    body = functools.partial(_edge_softmax_body, nheads)
    k = pl.kernel(
        body,
        out_type=[
            jax.ShapeDtypeStruct((E, 8), _f32),
            jax.ShapeDtypeStruct((2, NPAD, 8), _f32),
        ],
        mesh=_mesh(),
        scratch_types=[
            pltpu.VMEM((CHUNK,), _i32),
            pltpu.VMEM((CHUNK,), _i32),
            pltpu.VMEM((CHUNK, 16), _f32),
            pltpu.VMEM((CHUNK, 16), _f32),
            pltpu.VMEM((CHUNK, 8), _f32),
            pltpu.VMEM((16,), _f32),
            pltpu.VMEM_SHARED((NPAD, 8), _f32),
        ],
        compiler_params=pltpu.CompilerParams(needs_layout_passes=False, use_tc_tiling_on_sc=False),
        name=f"sc_edge_softmax_h{nheads}",
    )
    return k(src, tgt, asat, ubpad, z8)


def _edge_aggregate_body(nf, src_hbm, tgt_hbm, exp_hbm, rsum_hbm, h_hbm,
                         z_hbm, out_hbm,
                         src_v, tgt_v, ht, rs, expb, attb, shared_out):
    nheads = nf // NHID if nf == 64 else 1
    lg = {8: 3, 16: 4}[nf // nheads]
    c, s, ebase = _wid_base()
    n0 = s * ROWS_PT
    pltpu.sync_copy(z_hbm.at[pl.ds(n0, ROWS_PT)],
                    shared_out.at[pl.ds(n0, ROWS_PT)])
    plsc.subcore_barrier()

    qcols = [lax.shift_right_logical(_iota16() + 16 * q, lg)
             for q in range(nf // 16)]

    def chunk(t, _):
        e0 = ebase + t * CHUNK
        pltpu.sync_copy(src_hbm.at[pl.ds(e0, CHUNK)], src_v)
        pltpu.sync_copy(tgt_hbm.at[pl.ds(e0, CHUNK)], tgt_v)
        pltpu.sync_copy(h_hbm.at[tgt_v], ht)
        pltpu.sync_copy(rsum_hbm.at[src_v], rs)
        pltpu.sync_copy(exp_hbm.at[pl.ds(e0, CHUNK)], expb)

        def kb(k, _):
            rows = _iota16() + k * 16
            for h in range(nheads):
                a16 = (plsc.load_gather(expb, [rows, _splat(h)])
                       * plsc.load_gather(rs, [rows, _splat(h)]))
                plsc.store_scatter(attb, [rows, _splat(h)], a16)
            return 0

        lax.fori_loop(0, CHUNK // 16, kb, 0)

        def eb(i, _):
            ri = _splat(i)
            for q in range(nf // 16):
                a16 = plsc.load_gather(attb, [ri, qcols[q]])
                ht[i, pl.ds(16 * q, 16)] = a16 * ht[i, pl.ds(16 * q, 16)]
            return 0

        lax.fori_loop(0, CHUNK, eb, 0)
        pltpu.sync_copy(ht, shared_out.at[src_v], add=True)
        return 0

    lax.fori_loop(0, NCHUNK, chunk, 0)
    plsc.subcore_barrier()
    pltpu.sync_copy(shared_out.at[pl.ds(n0, ROWS_PT)],
                    out_hbm.at[c, pl.ds(n0, ROWS_PT)])


def _edge_aggregate(nf, src, tgt, expv, rsum, h, z):
    body = functools.partial(_edge_aggregate_body, nf)
    k = pl.kernel(
        body,
        out_type=jax.ShapeDtypeStruct((2, NPAD, nf), _f32),
        mesh=_mesh(),
        scratch_types=[
            pltpu.VMEM((CHUNK,), _i32),
            pltpu.VMEM((CHUNK,), _i32),
            pltpu.VMEM((CHUNK, nf), _f32),
            pltpu.VMEM((CHUNK, 16), _f32),
            pltpu.VMEM((CHUNK, 8), _f32),
            pltpu.VMEM((CHUNK, 8), _f32),
            pltpu.VMEM_SHARED((NPAD, nf), _f32),
        ],
        compiler_params=pltpu.CompilerParams(needs_layout_passes=False, use_tc_tiling_on_sc=False),
        name=f"sc_edge_aggregate_f{nf}",
    )
    return k(src, tgt, expv, rsum, h, z)


def _ub_from_mx(mx):
    m = jnp.max(mx, axis=0)
    ssum = m[:8] + m[8:]
    ub = jnp.maximum(ssum, ALPHA * ssum)
    return jnp.concatenate([ub, jnp.zeros((8,), _f32)])


def kernel(x, edge_index, W1, a1, b1, W2, a2, b2):
    src = edge_index[0]
    tgt = edge_index[1]

    w1cat = jnp.transpose(W1, (1, 0, 2)).reshape(D, NHEAD * NHID)
    eye8 = jnp.eye(NHEAD, dtype=_f32)
    a_src = (a1[:, :NHID, 0][:, :, None] * eye8[:, None, :]).reshape(
        NHEAD * NHID, NHEAD)
    a_tgt = (a1[:, NHID:, 0][:, :, None] * eye8[:, None, :]).reshape(
        NHEAD * NHID, NHEAD)
    a1cat = jnp.concatenate([a_src, a_tgt], axis=1)
    b1mat = b1[:, 0, :].reshape(1, NHEAD * NHID)
    w2m = W2[0]
    a2pad = jnp.concatenate(
        [a2[0, :NCLASS], jnp.zeros((NCLASS, 7), _f32),
         a2[0, NCLASS:], jnp.zeros((NCLASS, 7), _f32)], axis=1)
    b2mat = b2[0]

    z8 = jnp.zeros((NPAD, 8), _f32)
    z16 = jnp.zeros((NPAD, 16), _f32)
    z64 = jnp.zeros((NPAD, 64), _f32)

    h1, asat1, mx1 = _dense1(x, w1cat, a1cat)
    ub1 = _ub_from_mx(mx1)
    exp1, sum1p = _edge_softmax(NHEAD, src, tgt, asat1, ub1, z8)
    rsum1 = _rsum(sum1p)
    out1p = _edge_aggregate(64, src, tgt, exp1, rsum1, h1, z64)

    h2, asat2, mx2 = _dense2(out1p, b1mat, w2m, a2pad)
    ub2 = _ub_from_mx(mx2)
    exp2, sum2p = _edge_softmax(1, src, tgt, asat2, ub2, z8)
    rsum2 = _rsum(sum2p)
    out2p = _edge_aggregate(16, src, tgt, exp2, rsum2, h2, z16)

    return _final(out2p, b2mat)[:N]
